# Initial kernel scaffold; baseline (speedup 1.0000x reference)
import jax
import jax.numpy as jnp
from jax.experimental import pallas as pl


def kernel(embedding, embedding_decoder, enc0_w_x, enc0_w_h, enc0_gbias, dec_w_x, dec_w_hid, dec_w_h, dec_gbias, linear_w_t, linear_b, indices, lengths):
    raise NotImplementedError("write your pallas kernel here")


# speedup vs baseline: 1.2203x; 1.2203x over previous
import functools

import jax
import jax.numpy as jnp
from jax.experimental import pallas as pl
from jax.experimental.pallas import tpu as pltpu


def _round_up(x, m):
    return (x + m - 1) // m * m


def _fused_kernel(len_ref, xe_ref, xd_ref, ewx_ref, ewh_ref, egb_ref,
                  dwx_ref, dwh_ref, dwhid_ref, dgb_ref, wt_ref, b_ref,
                  hs_ref, logits_ref,
                  xw_scr, hs_tm_scr, hsbf_scr, *, ch, unroll):
    v = pl.program_id(1)
    Tp, bb, _ = xe_ref.shape
    hp = dwh_ref.shape[0]
    cdt = ewx_ref.dtype

    @pl.when(v == 0)
    def _recurrent_part():
        lens = len_ref[...]

        ewh = ewh_ref[...]
        h = jnp.zeros((bb, hp), jnp.float32)
        c = jnp.zeros((bb, hp), jnp.float32)
        hT = jnp.zeros((bb, hp), jnp.float32)
        for chunk in range(Tp // ch):
            xb = xe_ref[chunk * ch:(chunk + 1) * ch].reshape(ch * bb, -1)
            xw = jnp.dot(xb, ewx_ref[...], preferred_element_type=jnp.float32)
            xw_scr[...] = xw.reshape(ch, bb, 4 * hp) + egb_ref[...][None]

            def estep(s, carry):
                h, c, hT = carry
                gates = xw_scr[s] + jnp.dot(h.astype(cdt), ewh,
                                            preferred_element_type=jnp.float32)
                i_g = jax.nn.sigmoid(gates[:, 0 * hp:1 * hp])
                f_g = jax.nn.sigmoid(gates[:, 1 * hp:2 * hp])
                g_g = jnp.tanh(gates[:, 2 * hp:3 * hp])
                o_g = jax.nn.sigmoid(gates[:, 3 * hp:4 * hp])
                c_new = f_g * c + i_g * g_g
                h_new = o_g * jnp.tanh(c_new)
                t = chunk * ch + s
                hT = jnp.where((t + 1) == lens, h_new, hT)
                return h_new, c_new, hT

            h, c, hT = jax.lax.fori_loop(0, ch, estep, (h, c, hT),
                                         unroll=unroll)

        nrm = jnp.sqrt(jnp.sum(hT * hT, axis=1, keepdims=True))
        hidden = hT / jnp.where(nrm == 0.0, 1.0, nrm)
        hid_gates = jnp.dot(hidden, dwhid_ref[...],
                            preferred_element_type=jnp.float32)
        gbd = dgb_ref[...] + hid_gates

        dwh = dwh_ref[...]
        h = jnp.zeros((bb, hp), jnp.float32)
        c = jnp.zeros((bb, hp), jnp.float32)
        for chunk in range(Tp // ch):
            xb = xd_ref[chunk * ch:(chunk + 1) * ch].reshape(ch * bb, -1)
            xw = jnp.dot(xb, dwx_ref[...], preferred_element_type=jnp.float32)
            xw_scr[...] = xw.reshape(ch, bb, 4 * hp) + gbd[None]

            def dstep(s, carry):
                h, c = carry
                gates = xw_scr[s] + jnp.dot(h.astype(cdt), dwh,
                                            preferred_element_type=jnp.float32)
                i_g = jax.nn.sigmoid(gates[:, 0 * hp:1 * hp])
                f_g = jax.nn.sigmoid(gates[:, 1 * hp:2 * hp])
                g_g = jnp.tanh(gates[:, 2 * hp:3 * hp])
                o_g = jax.nn.sigmoid(gates[:, 3 * hp:4 * hp])
                c_new = f_g * c + i_g * g_g
                h_new = o_g * jnp.tanh(c_new)
                t = chunk * ch + s
                hs_tm_scr[t] = jnp.where(t < lens, h_new, 0.0)
                return h_new, c_new

            h, c = jax.lax.fori_loop(0, ch, dstep, (h, c), unroll=unroll)

        for t in range(Tp):
            row = hs_tm_scr[t]
            hs_ref[:, t, :] = row
            hsbf_scr[:, t, :] = row.astype(cdt)

    logits_ref[...] = (jnp.dot(hsbf_scr[...].reshape(bb * Tp, hp), wt_ref[...],
                               preferred_element_type=jnp.float32)
                       + b_ref[...])


def _seq2seq_fused(xe, xd, len_p, ewx, ewh, egb, dwx, dwh, dwhid, dgb,
                   wt, b, *, bb, tv=512, ch=32, unroll=4):
    Tp, Bp, Ep = xe.shape
    Hp = dwh.shape[0]
    Vp = wt.shape[-1]
    nb = Bp // bb
    nv = pl.cdiv(Vp, tv)
    ch = min(ch, Tp)
    assert Tp % ch == 0

    grid_spec = pltpu.PrefetchScalarGridSpec(
        num_scalar_prefetch=0,
        grid=(nb, nv),
        in_specs=[
            pl.BlockSpec((bb, 1), lambda i, v: (i, 0)),
            pl.BlockSpec((Tp, bb, Ep), lambda i, v: (0, i, 0)),
            pl.BlockSpec((Tp, bb, Ep), lambda i, v: (0, i, 0)),
            pl.BlockSpec(memory_space=pltpu.MemorySpace.VMEM),
            pl.BlockSpec(memory_space=pltpu.MemorySpace.VMEM),
            pl.BlockSpec(memory_space=pltpu.MemorySpace.VMEM),
            pl.BlockSpec(memory_space=pltpu.MemorySpace.VMEM),
            pl.BlockSpec(memory_space=pltpu.MemorySpace.VMEM),
            pl.BlockSpec(memory_space=pltpu.MemorySpace.VMEM),
            pl.BlockSpec(memory_space=pltpu.MemorySpace.VMEM),
            pl.BlockSpec((Hp, tv), lambda i, v: (0, v)),
            pl.BlockSpec((1, tv), lambda i, v: (0, v)),
        ],
        out_specs=[
            pl.BlockSpec((bb, Tp, Hp), lambda i, v: (i, 0, 0)),
            pl.BlockSpec((bb * Tp, tv), lambda i, v: (i, v)),
        ],
        scratch_shapes=[
            pltpu.VMEM((ch, bb, 4 * Hp), jnp.float32),
            pltpu.VMEM((Tp, bb, Hp), jnp.float32),
            pltpu.VMEM((bb, Tp, Hp), ewx.dtype),
        ])

    hs, logits = pl.pallas_call(
        functools.partial(_fused_kernel, ch=ch, unroll=unroll),
        out_shape=(jax.ShapeDtypeStruct((Bp, Tp, Hp), jnp.float32),
                   jax.ShapeDtypeStruct((Bp * Tp, Vp), jnp.float32)),
        grid_spec=grid_spec,
        compiler_params=pltpu.CompilerParams(
            dimension_semantics=("parallel", "arbitrary"),
            vmem_limit_bytes=63 << 20),
    )(len_p, xe, xd, ewx, ewh, egb, dwx, dwh, dwhid, dgb, wt, b)
    return hs, logits


def kernel(embedding, embedding_decoder, enc0_w_x, enc0_w_h, enc0_gbias,
           dec_w_x, dec_w_hid, dec_w_h, dec_gbias, linear_w_t, linear_b,
           indices, lengths):
    B, T = indices.shape
    Hp = dec_w_h.shape[0]
    V = embedding.shape[0]
    H = Hp
    cdt = linear_w_t.dtype

    if B > 8:
        Bp = _round_up(B, 16)
        bb = Bp // 2
    else:
        Bp = _round_up(B, 8)
        bb = Bp
    Tp = T if T <= 32 else _round_up(T, 32)

    idx_p = jnp.pad(indices.astype(jnp.int32), ((0, Bp - B), (0, Tp - T)))
    len_p = jnp.pad(lengths.astype(jnp.int32), (0, Bp - B)).reshape(Bp, 1)

    xe = jnp.take(embedding, idx_p.T, axis=0).astype(cdt)
    xd = jnp.take(embedding_decoder, idx_p.T, axis=0).astype(cdt)

    hs, logits = _seq2seq_fused(
        xe, xd, len_p, enc0_w_x, enc0_w_h, enc0_gbias,
        dec_w_x, dec_w_h, dec_w_hid, dec_gbias, linear_w_t, linear_b, bb=bb)

    decoded = logits.reshape(Bp, Tp, -1)[:B, :T, :V]
    emb_nhidden = hs[:B, :T, :H].reshape(B * T, H)
    return decoded, emb_nhidden
